# baseline (speedup 1.0000x reference)
import functools

import jax
import jax.numpy as jnp
from jax import lax
from jax.experimental import pallas as pl
from jax.experimental.pallas import tpu as pltpu
from jax.experimental.pallas import tpu_sc as plsc

_NC = 2
_NS = 16
_NW = _NC * _NS

_EMBED = 64
_CHUNK = 128
_GK = 4
_NBUF = 2


def _sc_gather_rows(table, idx2d, n_rows):
    rows_per_w = n_rows // _NW
    chunks_per_w = rows_per_w // _CHUNK
    n_groups = chunks_per_w // _GK
    group_rows = _GK * _CHUNK

    mesh = plsc.VectorSubcoreMesh(core_axis_name="c", subcore_axis_name="s")

    @functools.partial(
        pl.kernel,
        out_type=jax.ShapeDtypeStruct((n_rows, _EMBED), jnp.float32),
        mesh=mesh,
        scratch_types=[
            pltpu.VMEM((chunks_per_w, _CHUNK), jnp.int32),
            pltpu.VMEM((_NBUF, group_rows, _EMBED), jnp.float32),
            pltpu.SemaphoreType.DMA,
            pltpu.SemaphoreType.DMA,
        ],
        compiler_params=pltpu.CompilerParams(use_tc_tiling_on_sc=False),
    )
    def gather_kernel(table_hbm, idx_hbm, out_hbm, idx_v, rows_v, sem_g, sem_o):
        wid = lax.axis_index("s") * _NC + lax.axis_index("c")
        row_base = wid * rows_per_w
        pltpu.sync_copy(idx_hbm.at[pl.ds(wid * chunks_per_w, chunks_per_w)], idx_v)

        out_cps = [None] * _NBUF
        for g in range(n_groups):
            buf = g % _NBUF
            if g >= _NBUF:
                out_cps[buf].wait()
            cps = []
            for k in range(_GK):
                j = g * _GK + k
                cps.append(
                    pltpu.async_copy(
                        table_hbm.at[idx_v.at[j]],
                        rows_v.at[buf, pl.ds(k * _CHUNK, _CHUNK)],
                        sem_g,
                    )
                )
            for cp in cps:
                cp.wait()
            out_cps[buf] = pltpu.async_copy(
                rows_v.at[buf],
                out_hbm.at[pl.ds(row_base + g * group_rows, group_rows)],
                sem_o,
            )
        for buf in range(_NBUF):
            out_cps[buf].wait()

    return gather_kernel(table, idx2d)


def _tc_matmul(e, W, b2d, n_rows):
    blk = 8192

    def mm_body(e_ref, w_ref, b_ref, o_ref):
        o_ref[...] = (
            jnp.dot(e_ref[...], w_ref[...], preferred_element_type=jnp.float32)
            + b_ref[...]
        )

    return pl.pallas_call(
        mm_body,
        grid=(n_rows // blk,),
        in_specs=[
            pl.BlockSpec((blk, _EMBED), lambda i: (i, 0)),
            pl.BlockSpec((_EMBED, _EMBED), lambda i: (0, 0)),
            pl.BlockSpec((1, _EMBED), lambda i: (0, 0)),
        ],
        out_specs=pl.BlockSpec((blk, _EMBED), lambda i: (i, 0)),
        out_shape=jax.ShapeDtypeStruct((n_rows, _EMBED), jnp.float32),
    )(e, W, b2d)


def kernel(x, table, W, b):
    batch, fields = x.shape
    n_rows = batch * fields
    idx2d = x.astype(jnp.int32).reshape(n_rows // _CHUNK, _CHUNK)
    e = _sc_gather_rows(table, idx2d, n_rows)
    out = _tc_matmul(e, W, b.reshape(1, _EMBED), n_rows)
    return out.reshape(batch, fields, _EMBED)


# speedup vs baseline: 1.5819x; 1.5819x over previous
import functools

import jax
import jax.numpy as jnp
from jax import lax
from jax.experimental import pallas as pl
from jax.experimental.pallas import tpu as pltpu
from jax.experimental.pallas import tpu_sc as plsc

_NC = 2
_NS = 16
_NW = _NC * _NS

_EMBED = 64
_LANES = 128
_CHUNK = 128
_GK = 4
_NBUF = 2


def _sc_gather_rows(table, idx2d, n_rows):
    rows_per_w = n_rows // _NW
    chunks_per_w = rows_per_w // _CHUNK
    n_groups = chunks_per_w // _GK
    group_rows = _GK * _CHUNK

    mesh = plsc.VectorSubcoreMesh(core_axis_name="c", subcore_axis_name="s")

    @functools.partial(
        pl.kernel,
        out_type=jax.ShapeDtypeStruct((n_rows, _LANES), jnp.float32),
        mesh=mesh,
        scratch_types=[
            pltpu.VMEM((chunks_per_w, _CHUNK), jnp.int32),
            pltpu.VMEM((_NBUF, group_rows, _EMBED), jnp.float32),
            pltpu.SemaphoreType.DMA,
            pltpu.SemaphoreType.DMA,
        ],
        compiler_params=pltpu.CompilerParams(use_tc_tiling_on_sc=False),
    )
    def gather_kernel(table_hbm, idx_hbm, out_hbm, idx_v, rows_v, sem_g, sem_o):
        wid = lax.axis_index("s") * _NC + lax.axis_index("c")
        row_base = wid * rows_per_w
        pltpu.sync_copy(idx_hbm.at[pl.ds(wid * chunks_per_w, chunks_per_w)], idx_v)

        out_cps = [None] * _NBUF
        for g in range(n_groups):
            buf = g % _NBUF
            if g >= _NBUF:
                out_cps[buf].wait()
            cps = []
            for k in range(_GK):
                j = g * _GK + k
                cps.append(
                    pltpu.async_copy(
                        table_hbm.at[idx_v.at[j]],
                        rows_v.at[buf, pl.ds(k * _CHUNK, _CHUNK)],
                        sem_g,
                    )
                )
            for cp in cps:
                cp.wait()
            out_cps[buf] = pltpu.async_copy(
                rows_v.at[buf],
                out_hbm.at[
                    pl.ds(row_base + g * group_rows, group_rows), pl.ds(0, _EMBED)
                ],
                sem_o,
            )
        for buf in range(_NBUF):
            out_cps[buf].wait()

    return gather_kernel(table, idx2d)


def _tc_matmul(e, W, bcol, batch, fields):
    blk = 4096
    jb = batch // blk

    def mm_body(e_ref, w_ref, b_ref, o_ref):
        y = jax.lax.dot_general(
            w_ref[...],
            e_ref[:, 0:_EMBED],
            (((0,), (1,)), ((), ())),
            preferred_element_type=jnp.float32,
        )
        o_ref[...] = (y + b_ref[...]).reshape(1, _EMBED, blk)

    out_t = pl.pallas_call(
        mm_body,
        grid=(fields, jb),
        in_specs=[
            pl.BlockSpec((blk, _LANES), lambda f, j: (f * jb + j, 0)),
            pl.BlockSpec((_EMBED, _EMBED), lambda f, j: (0, 0)),
            pl.BlockSpec((_EMBED, 1), lambda f, j: (0, 0)),
        ],
        out_specs=pl.BlockSpec((1, _EMBED, blk), lambda f, j: (f, 0, j)),
        out_shape=jax.ShapeDtypeStruct((fields, _EMBED, batch), jnp.float32),
    )(e, W, bcol)
    return out_t.transpose(2, 0, 1)


def kernel(x, table, W, b):
    batch, fields = x.shape
    n_rows = batch * fields
    idx2d = x.astype(jnp.int32).T.reshape(n_rows // _CHUNK, _CHUNK)
    e = _sc_gather_rows(table, idx2d, n_rows)
    return _tc_matmul(e, W, b.reshape(_EMBED, 1), batch, fields)


# speedup vs baseline: 2.0984x; 1.3265x over previous
import functools

import jax
import jax.numpy as jnp
from jax import lax
from jax.experimental import pallas as pl
from jax.experimental.pallas import tpu as pltpu
from jax.experimental.pallas import tpu_sc as plsc

_NC = 2
_NS = 16
_NW = _NC * _NS

_EMBED = 64
_LANES = 128
_CHUNK = 128
_GK = 4
_NBUF = 2


def _tc_pack_table(tableT, vocab):
    blk = 4096

    def pack_body(a_ref, o_ref):
        ta = jnp.transpose(a_ref[...], (1, 0))
        o_ref[...] = jnp.concatenate([ta, ta], axis=1)

    return pl.pallas_call(
        pack_body,
        grid=(pl.cdiv(vocab, blk),),
        in_specs=[pl.BlockSpec((_EMBED, blk), lambda j: (0, j))],
        out_specs=pl.BlockSpec((blk, _LANES), lambda j: (j, 0)),
        out_shape=jax.ShapeDtypeStruct((vocab, _LANES), jnp.float32),
    )(tableT)


def _sc_gather_rows(t2, lines2d, n_rows):
    rows_per_w = n_rows // _NW
    chunks_per_w = rows_per_w // _CHUNK
    n_groups = chunks_per_w // _GK
    group_rows = _GK * _CHUNK

    mesh = plsc.VectorSubcoreMesh(core_axis_name="c", subcore_axis_name="s")

    @functools.partial(
        pl.kernel,
        out_type=jax.ShapeDtypeStruct((n_rows, _LANES), jnp.float32),
        mesh=mesh,
        scratch_types=[
            pltpu.VMEM((chunks_per_w, _CHUNK), jnp.int32),
            pltpu.VMEM((_NBUF, group_rows, _EMBED), jnp.float32),
            pltpu.SemaphoreType.DMA,
            pltpu.SemaphoreType.DMA,
        ],
        compiler_params=pltpu.CompilerParams(use_tc_tiling_on_sc=False),
    )
    def gather_kernel(t2_hbm, idx_hbm, out_hbm, idx_v, rows_v, sem_g, sem_o):
        wid = lax.axis_index("s") * _NC + lax.axis_index("c")
        row_base = wid * rows_per_w
        pltpu.sync_copy(idx_hbm.at[pl.ds(wid * chunks_per_w, chunks_per_w)], idx_v)

        out_cps = [None] * _NBUF
        for g in range(n_groups):
            buf = g % _NBUF
            if g >= _NBUF:
                out_cps[buf].wait()
            cps = []
            for k in range(_GK):
                j = g * _GK + k
                cps.append(
                    pltpu.async_copy(
                        t2_hbm.at[idx_v.at[j]],
                        rows_v.at[buf, pl.ds(k * _CHUNK, _CHUNK)],
                        sem_g,
                    )
                )
            for cp in cps:
                cp.wait()
            out_cps[buf] = pltpu.async_copy(
                rows_v.at[buf],
                out_hbm.at[
                    pl.ds(row_base + g * group_rows, group_rows), pl.ds(0, _EMBED)
                ],
                sem_o,
            )
        for buf in range(_NBUF):
            out_cps[buf].wait()

    return gather_kernel(t2, lines2d)


def _tc_matmul(e, W, bcol, batch, fields):
    blk = 4096
    jb = batch // blk

    def mm_body(e_ref, w_ref, b_ref, o_ref):
        y = lax.dot_general(
            w_ref[...],
            e_ref[:, 0:_EMBED],
            (((0,), (1,)), ((), ())),
            preferred_element_type=jnp.float32,
        )
        o_ref[...] = (y + b_ref[...]).reshape(1, _EMBED, blk)

    out_t = pl.pallas_call(
        mm_body,
        grid=(fields, jb),
        in_specs=[
            pl.BlockSpec((blk, _LANES), lambda f, j: (f * jb + j, 0)),
            pl.BlockSpec((_EMBED, _EMBED), lambda f, j: (0, 0)),
            pl.BlockSpec((_EMBED, 1), lambda f, j: (0, 0)),
        ],
        out_specs=pl.BlockSpec((1, _EMBED, blk), lambda f, j: (f, 0, j)),
        out_shape=jax.ShapeDtypeStruct((fields, _EMBED, batch), jnp.float32),
    )(e, W, bcol)
    return out_t.transpose(2, 0, 1)


def kernel(x, table, W, b):
    batch, fields = x.shape
    vocab = table.shape[0]
    n_rows = batch * fields

    tw = _tc_pack_table(table.T, vocab)
    t2 = tw.reshape(2 * vocab, _EMBED)

    lin = x.astype(jnp.int32).T.reshape(n_rows)
    lines2d = (lin * 2).reshape(n_rows // _CHUNK, _CHUNK)

    e = _sc_gather_rows(t2, lines2d, n_rows)
    return _tc_matmul(e, W, b.reshape(_EMBED, 1), batch, fields)


# speedup vs baseline: 2.3328x; 1.1117x over previous
import functools

import jax
import jax.numpy as jnp
from jax import lax
from jax.experimental import pallas as pl
from jax.experimental.pallas import tpu as pltpu
from jax.experimental.pallas import tpu_sc as plsc

_NC = 2
_NS = 16
_NW = _NC * _NS

_EMBED = 64
_LANES = 128
_CHUNK = 128
_GK = 4
_NBUF = 2


_PBLK = 4096


def _tc_pack_table(tableT, vocab):
    nblk = pl.cdiv(vocab, _PBLK)
    hb = _PBLK // 2

    def pack_body(a_ref, o_ref):
        ta = jnp.transpose(a_ref[...], (1, 0))
        o_ref[...] = jnp.concatenate([ta[0:hb], ta[hb:_PBLK]], axis=1)

    return pl.pallas_call(
        pack_body,
        grid=(nblk,),
        in_specs=[pl.BlockSpec((_EMBED, _PBLK), lambda j: (0, j))],
        out_specs=pl.BlockSpec((hb, _LANES), lambda j: (j, 0)),
        out_shape=jax.ShapeDtypeStruct((nblk * hb, _LANES), jnp.float32),
    )(tableT)


def _sc_gather_rows(t2, lines2d, n_rows):
    rows_per_w = n_rows // _NW
    chunks_per_w = rows_per_w // _CHUNK
    n_groups = chunks_per_w // _GK
    group_rows = _GK * _CHUNK

    mesh = plsc.VectorSubcoreMesh(core_axis_name="c", subcore_axis_name="s")

    @functools.partial(
        pl.kernel,
        out_type=jax.ShapeDtypeStruct((n_rows, _LANES), jnp.float32),
        mesh=mesh,
        scratch_types=[
            pltpu.VMEM((chunks_per_w, _CHUNK), jnp.int32),
            pltpu.VMEM((_NBUF, group_rows, _EMBED), jnp.float32),
            pltpu.SemaphoreType.DMA,
            pltpu.SemaphoreType.DMA,
        ],
        compiler_params=pltpu.CompilerParams(use_tc_tiling_on_sc=False),
    )
    def gather_kernel(t2_hbm, idx_hbm, out_hbm, idx_v, rows_v, sem_g, sem_o):
        wid = lax.axis_index("s") * _NC + lax.axis_index("c")
        row_base = wid * rows_per_w
        pltpu.sync_copy(idx_hbm.at[pl.ds(wid * chunks_per_w, chunks_per_w)], idx_v)

        out_cps = [None] * _NBUF
        for g in range(n_groups):
            buf = g % _NBUF
            if g >= _NBUF:
                out_cps[buf].wait()
            cps = []
            for k in range(_GK):
                j = g * _GK + k
                cps.append(
                    pltpu.async_copy(
                        t2_hbm.at[idx_v.at[j]],
                        rows_v.at[buf, pl.ds(k * _CHUNK, _CHUNK)],
                        sem_g,
                    )
                )
            for cp in cps:
                cp.wait()
            out_cps[buf] = pltpu.async_copy(
                rows_v.at[buf],
                out_hbm.at[
                    pl.ds(row_base + g * group_rows, group_rows), pl.ds(0, _EMBED)
                ],
                sem_o,
            )
        for buf in range(_NBUF):
            out_cps[buf].wait()

    return gather_kernel(t2, lines2d)


def _tc_matmul(e, W, bcol, batch, fields):
    blk = 4096
    jb = batch // blk

    def mm_body(e_ref, w_ref, b_ref, o_ref):
        y = lax.dot_general(
            w_ref[...],
            e_ref[:, 0:_EMBED],
            (((0,), (1,)), ((), ())),
            preferred_element_type=jnp.float32,
        )
        o_ref[...] = (y + b_ref[...]).reshape(1, _EMBED, blk)

    out_t = pl.pallas_call(
        mm_body,
        grid=(fields, jb),
        in_specs=[
            pl.BlockSpec((blk, _LANES), lambda f, j: (f * jb + j, 0)),
            pl.BlockSpec((_EMBED, _EMBED), lambda f, j: (0, 0)),
            pl.BlockSpec((_EMBED, 1), lambda f, j: (0, 0)),
        ],
        out_specs=pl.BlockSpec((1, _EMBED, blk), lambda f, j: (f, 0, j)),
        out_shape=jax.ShapeDtypeStruct((fields, _EMBED, batch), jnp.float32),
    )(e, W, bcol)
    return out_t.transpose(2, 0, 1)


def kernel(x, table, W, b):
    batch, fields = x.shape
    vocab = table.shape[0]
    n_rows = batch * fields

    tw = _tc_pack_table(table.T, vocab)
    t2 = tw.reshape(tw.shape[0] * 2, _EMBED)

    lin = x.astype(jnp.int32).T.reshape(n_rows)
    hb = _PBLK // 2
    rows2 = (
        (lin // _PBLK) * _PBLK
        + ((lin % hb) * 2)
        + ((lin % _PBLK) // hb)
    )
    lines2d = rows2.reshape(n_rows // _CHUNK, _CHUNK)

    e = _sc_gather_rows(t2, lines2d, n_rows)
    return _tc_matmul(e, W, b.reshape(_EMBED, 1), batch, fields)


# speedup vs baseline: 2.8189x; 1.2084x over previous
import functools

import jax
import jax.numpy as jnp
from jax import lax
from jax.experimental import pallas as pl
from jax.experimental.pallas import tpu as pltpu
from jax.experimental.pallas import tpu_sc as plsc

_NC = 2
_NS = 16
_NW = _NC * _NS

_EMBED = 64
_LANES = 128
_CHUNK = 128
_GK = 4
_NBUF = 2


_PBLK = 8192


def _tc_pack_table(tableT, vocab):
    nblk = pl.cdiv(vocab, _PBLK)
    hb = _PBLK // 2

    def pack_body(a_ref, o_ref):
        ta = jnp.transpose(a_ref[...], (1, 0))
        o_ref[...] = jnp.concatenate([ta[0:hb], ta[hb:_PBLK]], axis=1)

    return pl.pallas_call(
        pack_body,
        grid=(nblk,),
        in_specs=[pl.BlockSpec((_EMBED, _PBLK), lambda j: (0, j))],
        out_specs=pl.BlockSpec((hb, _LANES), lambda j: (j, 0)),
        out_shape=jax.ShapeDtypeStruct((nblk * hb, _LANES), jnp.float32),
    )(tableT)


def _sc_gather_rows(t2, lines2d, n_rows):
    rows_per_w = n_rows // _NW
    chunks_per_w = rows_per_w // _CHUNK
    n_groups = chunks_per_w // _GK
    group_rows = _GK * _CHUNK

    mesh = plsc.VectorSubcoreMesh(core_axis_name="c", subcore_axis_name="s")

    @functools.partial(
        pl.kernel,
        out_type=jax.ShapeDtypeStruct((n_rows, _LANES), jnp.float32),
        mesh=mesh,
        scratch_types=[
            pltpu.VMEM((chunks_per_w, _CHUNK), jnp.int32),
            pltpu.VMEM((_NBUF, group_rows, _EMBED), jnp.float32),
            pltpu.SemaphoreType.DMA,
            pltpu.SemaphoreType.DMA,
        ],
        compiler_params=pltpu.CompilerParams(use_tc_tiling_on_sc=False),
    )
    def gather_kernel(t2_hbm, idx_hbm, out_hbm, idx_v, rows_v, sem_g, sem_o):
        wid = lax.axis_index("s") * _NC + lax.axis_index("c")
        row_base = wid * rows_per_w
        pltpu.sync_copy(idx_hbm.at[pl.ds(wid * chunks_per_w, chunks_per_w)], idx_v)

        out_cps = [None] * _NBUF
        for g in range(n_groups):
            buf = g % _NBUF
            if g >= _NBUF:
                out_cps[buf].wait()
            cps = []
            for k in range(_GK):
                j = g * _GK + k
                cps.append(
                    pltpu.async_copy(
                        t2_hbm.at[idx_v.at[j]],
                        rows_v.at[buf, pl.ds(k * _CHUNK, _CHUNK)],
                        sem_g,
                    )
                )
            for cp in cps:
                cp.wait()
            out_cps[buf] = pltpu.async_copy(
                rows_v.at[buf],
                out_hbm.at[
                    pl.ds(row_base + g * group_rows, group_rows), pl.ds(0, _EMBED)
                ],
                sem_o,
            )
        for buf in range(_NBUF):
            out_cps[buf].wait()

    return gather_kernel(t2, lines2d)


def _tc_matmul(e, W, bcol, batch, fields):
    blk = 8192
    jb = batch // blk

    def mm_body(e_ref, w_ref, b_ref, o_ref):
        y = lax.dot_general(
            w_ref[...],
            e_ref[:, 0:_EMBED],
            (((0,), (1,)), ((), ())),
            preferred_element_type=jnp.float32,
        )
        o_ref[...] = (y + b_ref[...]).reshape(1, _EMBED, blk)

    out_t = pl.pallas_call(
        mm_body,
        grid=(fields, jb),
        in_specs=[
            pl.BlockSpec((blk, _LANES), lambda f, j: (f * jb + j, 0)),
            pl.BlockSpec((_EMBED, _EMBED), lambda f, j: (0, 0)),
            pl.BlockSpec((_EMBED, 1), lambda f, j: (0, 0)),
        ],
        out_specs=pl.BlockSpec((1, _EMBED, blk), lambda f, j: (f, 0, j)),
        out_shape=jax.ShapeDtypeStruct((fields, _EMBED, batch), jnp.float32),
    )(e, W, bcol)
    return out_t.transpose(2, 0, 1)


def kernel(x, table, W, b):
    batch, fields = x.shape
    vocab = table.shape[0]
    n_rows = batch * fields

    tw = _tc_pack_table(table.T, vocab)
    t2 = tw.reshape(tw.shape[0] * 2, _EMBED)

    lin = x.astype(jnp.int32).T.reshape(n_rows)
    hb = _PBLK // 2
    rows2 = (
        (lin // _PBLK) * _PBLK
        + ((lin % hb) * 2)
        + ((lin % _PBLK) // hb)
    )
    lines2d = rows2.reshape(n_rows // _CHUNK, _CHUNK)

    e = _sc_gather_rows(t2, lines2d, n_rows)
    return _tc_matmul(e, W, b.reshape(_EMBED, 1), batch, fields)


# speedup vs baseline: 3.0988x; 1.0993x over previous
import functools

import jax
import jax.numpy as jnp
from jax import lax
from jax.experimental import pallas as pl
from jax.experimental.pallas import tpu as pltpu
from jax.experimental.pallas import tpu_sc as plsc

_NC = 2
_NS = 16
_NW = _NC * _NS

_EMBED = 64
_LANES = 128
_CHUNK = 128
_GK = 4
_NBUF = 2


_PBLK = 16384


def _tc_pack_table(tableT, vocab):
    nblk = pl.cdiv(vocab, _PBLK)
    hb = _PBLK // 2

    def pack_body(a_ref, o_ref):
        ta = jnp.transpose(a_ref[...], (1, 0))
        o_ref[...] = jnp.concatenate([ta[0:hb], ta[hb:_PBLK]], axis=1)

    return pl.pallas_call(
        pack_body,
        grid=(nblk,),
        in_specs=[pl.BlockSpec((_EMBED, _PBLK), lambda j: (0, j))],
        out_specs=pl.BlockSpec((hb, _LANES), lambda j: (j, 0)),
        out_shape=jax.ShapeDtypeStruct((nblk * hb, _LANES), jnp.float32),
    )(tableT)


def _sc_gather_rows(t2, lines2d, n_rows):
    rows_per_w = n_rows // _NW
    chunks_per_w = rows_per_w // _CHUNK
    n_groups = chunks_per_w // _GK
    group_rows = _GK * _CHUNK

    mesh = plsc.VectorSubcoreMesh(core_axis_name="c", subcore_axis_name="s")

    @functools.partial(
        pl.kernel,
        out_type=jax.ShapeDtypeStruct((n_rows, _LANES), jnp.float32),
        mesh=mesh,
        scratch_types=[
            pltpu.VMEM((chunks_per_w, _CHUNK), jnp.int32),
            pltpu.VMEM((_NBUF, group_rows, _EMBED), jnp.float32),
            pltpu.SemaphoreType.DMA,
            pltpu.SemaphoreType.DMA,
        ],
        compiler_params=pltpu.CompilerParams(use_tc_tiling_on_sc=False),
    )
    def gather_kernel(t2_hbm, idx_hbm, out_hbm, idx_v, rows_v, sem_g, sem_o):
        wid = lax.axis_index("s") * _NC + lax.axis_index("c")
        row_base = wid * rows_per_w
        pltpu.sync_copy(idx_hbm.at[pl.ds(wid * chunks_per_w, chunks_per_w)], idx_v)

        out_cps = [None] * _NBUF
        for g in range(n_groups):
            buf = g % _NBUF
            if g >= _NBUF:
                out_cps[buf].wait()
            cps = []
            for k in range(_GK):
                j = g * _GK + k
                cps.append(
                    pltpu.async_copy(
                        t2_hbm.at[idx_v.at[j]],
                        rows_v.at[buf, pl.ds(k * _CHUNK, _CHUNK)],
                        sem_g,
                    )
                )
            for cp in cps:
                cp.wait()
            out_cps[buf] = pltpu.async_copy(
                rows_v.at[buf],
                out_hbm.at[
                    pl.ds(row_base + g * group_rows, group_rows), pl.ds(0, _EMBED)
                ],
                sem_o,
            )
        for buf in range(_NBUF):
            out_cps[buf].wait()

    return gather_kernel(t2, lines2d)


def _tc_matmul(e, W, bcol, batch, fields):
    blk = 16384
    jb = batch // blk

    def mm_body(e_ref, w_ref, b_ref, o_ref):
        y = lax.dot_general(
            w_ref[...],
            e_ref[:, 0:_EMBED],
            (((0,), (1,)), ((), ())),
            preferred_element_type=jnp.float32,
        )
        o_ref[...] = (y + b_ref[...]).reshape(1, _EMBED, blk)

    out_t = pl.pallas_call(
        mm_body,
        grid=(fields, jb),
        in_specs=[
            pl.BlockSpec((blk, _LANES), lambda f, j: (f * jb + j, 0)),
            pl.BlockSpec((_EMBED, _EMBED), lambda f, j: (0, 0)),
            pl.BlockSpec((_EMBED, 1), lambda f, j: (0, 0)),
        ],
        out_specs=pl.BlockSpec((1, _EMBED, blk), lambda f, j: (f, 0, j)),
        out_shape=jax.ShapeDtypeStruct((fields, _EMBED, batch), jnp.float32),
    )(e, W, bcol)
    return out_t.transpose(2, 0, 1)


def kernel(x, table, W, b):
    batch, fields = x.shape
    vocab = table.shape[0]
    n_rows = batch * fields

    tw = _tc_pack_table(table.T, vocab)
    t2 = tw.reshape(tw.shape[0] * 2, _EMBED)

    lin = x.astype(jnp.int32).T.reshape(n_rows)
    hb = _PBLK // 2
    rows2 = (
        (lin // _PBLK) * _PBLK
        + ((lin % hb) * 2)
        + ((lin % _PBLK) // hb)
    )
    lines2d = rows2.reshape(n_rows // _CHUNK, _CHUNK)

    e = _sc_gather_rows(t2, lines2d, n_rows)
    return _tc_matmul(e, W, b.reshape(_EMBED, 1), batch, fields)


# speedup vs baseline: 3.1997x; 1.0326x over previous
import functools

import jax
import jax.numpy as jnp
from jax import lax
from jax.experimental import pallas as pl
from jax.experimental.pallas import tpu as pltpu
from jax.experimental.pallas import tpu_sc as plsc

_NC = 2
_NS = 16
_NW = _NC * _NS

_EMBED = 64
_LANES = 128
_CHUNK = 128
_GK = 4
_NBUF = 2


_PBLK = 32768


def _tc_pack_table(tableT, vocab):
    nblk = pl.cdiv(vocab, _PBLK)
    hb = _PBLK // 2

    def pack_body(a_ref, o_ref):
        ta = jnp.transpose(a_ref[...], (1, 0))
        o_ref[...] = jnp.concatenate([ta[0:hb], ta[hb:_PBLK]], axis=1)

    return pl.pallas_call(
        pack_body,
        grid=(nblk,),
        in_specs=[pl.BlockSpec((_EMBED, _PBLK), lambda j: (0, j))],
        out_specs=pl.BlockSpec((hb, _LANES), lambda j: (j, 0)),
        out_shape=jax.ShapeDtypeStruct((nblk * hb, _LANES), jnp.float32),
        compiler_params=pltpu.CompilerParams(vmem_limit_bytes=58 * 1024 * 1024),
    )(tableT)


def _sc_gather_rows(t2, lines2d, n_rows):
    rows_per_w = n_rows // _NW
    chunks_per_w = rows_per_w // _CHUNK
    n_groups = chunks_per_w // _GK
    group_rows = _GK * _CHUNK

    mesh = plsc.VectorSubcoreMesh(core_axis_name="c", subcore_axis_name="s")

    @functools.partial(
        pl.kernel,
        out_type=jax.ShapeDtypeStruct((n_rows, _LANES), jnp.float32),
        mesh=mesh,
        scratch_types=[
            pltpu.VMEM((chunks_per_w, _CHUNK), jnp.int32),
            pltpu.VMEM((_NBUF, group_rows, _EMBED), jnp.float32),
            pltpu.SemaphoreType.DMA,
            pltpu.SemaphoreType.DMA,
        ],
        compiler_params=pltpu.CompilerParams(use_tc_tiling_on_sc=False),
    )
    def gather_kernel(t2_hbm, idx_hbm, out_hbm, idx_v, rows_v, sem_g, sem_o):
        wid = lax.axis_index("s") * _NC + lax.axis_index("c")
        row_base = wid * rows_per_w
        pltpu.sync_copy(idx_hbm.at[pl.ds(wid * chunks_per_w, chunks_per_w)], idx_v)

        out_cps = [None] * _NBUF
        for g in range(n_groups):
            buf = g % _NBUF
            if g >= _NBUF:
                out_cps[buf].wait()
            cps = []
            for k in range(_GK):
                j = g * _GK + k
                cps.append(
                    pltpu.async_copy(
                        t2_hbm.at[idx_v.at[j]],
                        rows_v.at[buf, pl.ds(k * _CHUNK, _CHUNK)],
                        sem_g,
                    )
                )
            for cp in cps:
                cp.wait()
            out_cps[buf] = pltpu.async_copy(
                rows_v.at[buf],
                out_hbm.at[
                    pl.ds(row_base + g * group_rows, group_rows), pl.ds(0, _EMBED)
                ],
                sem_o,
            )
        for buf in range(_NBUF):
            out_cps[buf].wait()

    return gather_kernel(t2, lines2d)


def _tc_matmul(e, W, bcol, batch, fields):
    fpb = 2

    def mm_body(e_ref, w_ref, b_ref, o_ref):
        y = lax.dot_general(
            w_ref[...],
            e_ref[:, 0:_EMBED],
            (((0,), (1,)), ((), ())),
            preferred_element_type=jnp.float32,
        ) + b_ref[...]
        for i in range(fpb):
            o_ref[i, :, :] = y[:, i * batch:(i + 1) * batch]

    out_t = pl.pallas_call(
        mm_body,
        grid=(fields // fpb,),
        in_specs=[
            pl.BlockSpec((fpb * batch, _LANES), lambda f: (f, 0)),
            pl.BlockSpec((_EMBED, _EMBED), lambda f: (0, 0)),
            pl.BlockSpec((_EMBED, 1), lambda f: (0, 0)),
        ],
        out_specs=pl.BlockSpec((fpb, _EMBED, batch), lambda f: (f, 0, 0)),
        out_shape=jax.ShapeDtypeStruct((fields, _EMBED, batch), jnp.float32),
        compiler_params=pltpu.CompilerParams(vmem_limit_bytes=58 * 1024 * 1024),
    )(e, W, bcol)
    return out_t.transpose(2, 0, 1)


def kernel(x, table, W, b):
    batch, fields = x.shape
    vocab = table.shape[0]
    n_rows = batch * fields

    tw = _tc_pack_table(table.T, vocab)
    t2 = tw.reshape(tw.shape[0] * 2, _EMBED)

    lin = x.astype(jnp.int32).T.reshape(n_rows)
    hb = _PBLK // 2
    rows2 = (
        (lin // _PBLK) * _PBLK
        + ((lin % hb) * 2)
        + ((lin % _PBLK) // hb)
    )
    lines2d = rows2.reshape(n_rows // _CHUNK, _CHUNK)

    e = _sc_gather_rows(t2, lines2d, n_rows)
    return _tc_matmul(e, W, b.reshape(_EMBED, 1), batch, fields)
